# baseline (speedup 1.0000x reference)
import functools

import jax
import jax.numpy as jnp
from jax import lax
from jax.experimental import pallas as pl
from jax.experimental.pallas import tpu as pltpu
from jax.experimental.pallas import tpu_sc as plsc

PRE_SEQ_LEN = 128
EMBED_DIM = 49152
K = 8
DC = EMBED_DIM // K
NW = 32
R = 8


def _sc_gather(table2, eidx, bk):
    rows_per_w = bk // NW
    nbatch = rows_per_w // R
    mesh = plsc.VectorSubcoreMesh(core_axis_name="c", subcore_axis_name="s")

    @functools.partial(
        pl.kernel,
        mesh=mesh,
        out_type=jax.ShapeDtypeStruct((bk, DC), jnp.float32),
        scratch_types=[
            pltpu.VMEM((rows_per_w,), jnp.int32),
            pltpu.VMEM((R, DC), jnp.float32),
            pltpu.SemaphoreType.DMA,
        ],
    )
    def body(table_hbm, eidx_hbm, out_hbm, idx_v, buf, sem):
        wid = lax.axis_index("s") * 2 + lax.axis_index("c")
        base = wid * rows_per_w
        pltpu.sync_copy(eidx_hbm.at[pl.ds(base, rows_per_w)], idx_v)

        def step(j, carry):
            off = pl.multiple_of(j * R, 8)
            pltpu.async_copy(
                table_hbm.at[idx_v.at[pl.ds(off, R)]], buf, sem
            ).wait()
            pltpu.sync_copy(buf, out_hbm.at[pl.ds(base + off, R)])
            return carry

        lax.fori_loop(0, nbatch, step, 0)

    return body(table2, eidx)


def kernel(prefix, embedding):
    b, s = prefix.shape
    bk = b * s * K
    idx = prefix.reshape(-1).astype(jnp.int32)
    eidx = (idx[:, None] * K + jnp.arange(K, dtype=jnp.int32)[None, :]).reshape(-1)
    table2 = embedding.reshape(PRE_SEQ_LEN * K, DC)
    out2 = _sc_gather(table2, eidx, bk)
    return out2.reshape(b, s, EMBED_DIM)
